# baseline (speedup 1.0000x reference)
import functools

import jax
import jax.numpy as jnp
from jax import lax
from jax.experimental import pallas as pl
from jax.experimental.pallas import tpu as pltpu
from jax.experimental.pallas import tpu_sc as plsc

B = 32
T = 1024
V = 8192
D = 256
N = B * T

NC = 2
NS = 16
NW = NC * NS

B_PER_W = N // NW
CHUNK = 128
NCHUNK = B_PER_W // CHUNK
NBUF = 3

_mesh = plsc.VectorSubcoreMesh(core_axis_name="c", subcore_axis_name="s")


@functools.partial(
    pl.kernel,
    mesh=_mesh,
    out_type=jax.ShapeDtypeStruct((N, D), jnp.float32),
    scratch_types=[
        pltpu.VMEM((B_PER_W,), jnp.int32),
        *[pltpu.VMEM((CHUNK, D), jnp.float32) for _ in range(NBUF)],
        *[pltpu.SemaphoreType.DMA for _ in range(2 * NBUF)],
    ],
)
def _gather_kernel(idx_hbm, table_hbm, out_hbm, idx_v, *scratch):
    bufs = scratch[:NBUF]
    gsems = scratch[NBUF:2 * NBUF]
    ssems = scratch[2 * NBUF:]
    wid = lax.axis_index("s") * NC + lax.axis_index("c")
    base = wid * B_PER_W
    pltpu.sync_copy(idx_hbm.at[wid], idx_v)
    gcp = [None] * NBUF
    scp = [None] * NBUF
    for j in range(NCHUNK + 1):
        if j < NCHUNK:
            b = j % NBUF
            if j >= NBUF:
                scp[b].wait()
            gcp[b] = pltpu.async_copy(
                table_hbm.at[idx_v.at[pl.ds(j * CHUNK, CHUNK)]], bufs[b], gsems[b]
            )
        if j >= 1:
            i = j - 1
            b = i % NBUF
            gcp[b].wait()
            scp[b] = pltpu.async_copy(
                bufs[b], out_hbm.at[pl.ds(base + i * CHUNK, CHUNK)], ssems[b]
            )
    for i in range(max(0, NCHUNK - NBUF), NCHUNK):
        scp[i % NBUF].wait()


def kernel(inputs, codebook):
    out = _gather_kernel(inputs.astype(jnp.int32), codebook)
    return out.reshape(B, T, D)


# speedup vs baseline: 1.0036x; 1.0036x over previous
import functools

import jax
import jax.numpy as jnp
from jax import lax
from jax.experimental import pallas as pl
from jax.experimental.pallas import tpu as pltpu
from jax.experimental.pallas import tpu_sc as plsc

B = 32
T = 1024
V = 8192
D = 256
N = B * T

NC = 2
NS = 16
NW = NC * NS

B_PER_W = N // NW
CHUNK = 64
NCHUNK = B_PER_W // CHUNK
NBUF = 6

_mesh = plsc.VectorSubcoreMesh(core_axis_name="c", subcore_axis_name="s")


@functools.partial(
    pl.kernel,
    mesh=_mesh,
    out_type=jax.ShapeDtypeStruct((N, D), jnp.float32),
    scratch_types=[
        pltpu.VMEM((B_PER_W,), jnp.int32),
        *[pltpu.VMEM((CHUNK, D), jnp.float32) for _ in range(NBUF)],
        *[pltpu.SemaphoreType.DMA for _ in range(2 * NBUF)],
    ],
)
def _gather_kernel(idx_hbm, table_hbm, out_hbm, idx_v, *scratch):
    bufs = scratch[:NBUF]
    gsems = scratch[NBUF:2 * NBUF]
    ssems = scratch[2 * NBUF:]
    wid = lax.axis_index("s") * NC + lax.axis_index("c")
    base = wid * B_PER_W
    pltpu.sync_copy(idx_hbm.at[wid], idx_v)
    gcp = [None] * NBUF
    scp = [None] * NBUF
    for j in range(NCHUNK + 1):
        if j < NCHUNK:
            b = j % NBUF
            if j >= NBUF:
                scp[b].wait()
            gcp[b] = pltpu.async_copy(
                table_hbm.at[idx_v.at[pl.ds(j * CHUNK, CHUNK)]], bufs[b], gsems[b]
            )
        if j >= 1:
            i = j - 1
            b = i % NBUF
            gcp[b].wait()
            scp[b] = pltpu.async_copy(
                bufs[b], out_hbm.at[pl.ds(base + i * CHUNK, CHUNK)], ssems[b]
            )
    for i in range(max(0, NCHUNK - NBUF), NCHUNK):
        scp[i % NBUF].wait()


def kernel(inputs, codebook):
    out = _gather_kernel(inputs.astype(jnp.int32), codebook)
    return out.reshape(B, T, D)


# speedup vs baseline: 1.0054x; 1.0017x over previous
import functools

import jax
import jax.numpy as jnp
from jax import lax
from jax.experimental import pallas as pl
from jax.experimental.pallas import tpu as pltpu
from jax.experimental.pallas import tpu_sc as plsc

B = 32
T = 1024
V = 8192
D = 256
N = B * T

NC = 2
NS = 16
NW = NC * NS

B_PER_W = N // NW
CHUNK = 64
NCHUNK = B_PER_W // CHUNK
NBUF = 6

_mesh = plsc.VectorSubcoreMesh(core_axis_name="c", subcore_axis_name="s")


@functools.partial(
    pl.kernel,
    mesh=_mesh,
    out_type=jax.ShapeDtypeStruct((NW, B_PER_W, D), jnp.float32),
    scratch_types=[
        pltpu.VMEM((B_PER_W,), jnp.int32),
        *[pltpu.VMEM((CHUNK, D), jnp.float32) for _ in range(NBUF)],
        *[pltpu.SemaphoreType.DMA for _ in range(2 * NBUF)],
    ],
)
def _gather_kernel(idx_hbm, table_hbm, out_hbm, idx_v, *scratch):
    bufs = scratch[:NBUF]
    gsems = scratch[NBUF:2 * NBUF]
    ssems = scratch[2 * NBUF:]
    wid = lax.axis_index("s") * NC + lax.axis_index("c")
    pltpu.sync_copy(idx_hbm.at[wid], idx_v)
    gcp = [None] * NBUF
    scp = [None] * NBUF
    for j in range(NCHUNK + 1):
        if j < NCHUNK:
            b = j % NBUF
            if j >= NBUF:
                scp[b].wait()
            gcp[b] = pltpu.async_copy(
                table_hbm.at[idx_v.at[pl.ds(j * CHUNK, CHUNK)]], bufs[b], gsems[b]
            )
        if j >= 1:
            i = j - 1
            b = i % NBUF
            gcp[b].wait()
            scp[b] = pltpu.async_copy(
                bufs[b], out_hbm.at[wid, pl.ds(i * CHUNK, CHUNK)], ssems[b]
            )
    for i in range(max(0, NCHUNK - NBUF), NCHUNK):
        scp[i % NBUF].wait()


def kernel(inputs, codebook):
    return _gather_kernel(inputs, codebook)
